# baseline (speedup 1.0000x reference)
import functools

import jax
import jax.numpy as jnp
from jax import lax
from jax.experimental import pallas as pl
from jax.experimental.pallas import tpu as pltpu
from jax.experimental.pallas import tpu_sc as plsc

E = 8
H = 2048
F = 1408
T = 2048
TM = 256
NJ = (2 * T) // TM + E
NS = NJ * TM


def _router_body(x_ref, rwt_ref, pos_ref, wexp_ref, toff_ref):
    x = x_ref[...]
    logits = jnp.dot(x, rwt_ref[...], preferred_element_type=jnp.float32)
    m = jnp.max(logits, axis=-1, keepdims=True)
    ex = jnp.exp(logits - m)
    aff = ex / jnp.sum(ex, axis=-1, keepdims=True)
    idx = lax.broadcasted_iota(jnp.int32, aff.shape, 1)
    m1 = jnp.max(aff, axis=-1, keepdims=True)
    i1 = jnp.min(jnp.where(aff == m1, idx, E), axis=-1, keepdims=True)
    aff2 = jnp.where(idx == i1, -1.0, aff)
    m2 = jnp.max(aff2, axis=-1, keepdims=True)
    i2 = jnp.min(jnp.where(aff2 == m2, idx, E), axis=-1, keepdims=True)
    s = m1 + m2
    wexp_ref[0] = jnp.broadcast_to(m1 / s, (T, 128))
    wexp_ref[1] = jnp.broadcast_to(m2 / s, (T, 128))

    oh1 = (idx == i1).astype(jnp.float32)
    oh2 = (idx == i2).astype(jnp.float32)
    tri = (lax.broadcasted_iota(jnp.int32, (128, 128), 0)
           >= lax.broadcasted_iota(jnp.int32, (128, 128), 1)).astype(jnp.float32)

    def _cumsum_tokens(oh):
        outs = []
        prefix = jnp.zeros((1, E), jnp.float32)
        for blk in range(T // 128):
            part = oh[blk * 128:(blk + 1) * 128, :]
            cw = jnp.dot(tri, part, preferred_element_type=jnp.float32) + prefix
            outs.append(cw)
            prefix = cw[127:128, :]
        return jnp.concatenate(outs, axis=0)

    c1 = _cumsum_tokens(oh1)
    c2 = _cumsum_tokens(oh2)
    n1 = c1[T - 1:T, :]
    counts = n1 + c2[T - 1:T, :]
    nt = jnp.ceil(counts / TM)
    lane = lax.broadcasted_iota(jnp.int32, (E, E), 0)
    lane2 = lax.broadcasted_iota(jnp.int32, (E, E), 1)
    strict_lt = (lane < lane2).astype(jnp.float32)
    toff = jnp.dot(nt, strict_lt, preferred_element_type=jnp.float32)
    ntot = jnp.sum(nt, axis=-1, keepdims=True)
    off_pad = toff * TM
    pos1 = jnp.sum(oh1 * (off_pad + c1 - oh1), axis=-1, keepdims=True)
    pos2 = jnp.sum(oh2 * (off_pad + n1 + c2 - oh2), axis=-1, keepdims=True)
    pos_pair = jnp.concatenate([pos1, pos2], axis=1).astype(jnp.int32)
    pos_ref[...] = pos_pair.T
    toff_ref[...] = jnp.concatenate(
        [toff, jnp.broadcast_to(ntot, (1, E))], axis=1).astype(jnp.int32)


def _dispatch_sc(x_hbm, pos_hbm, wexp_hbm, xs_hbm, ws_hbm,
                 idxw0_v, idxw1_v, w0_v, w1_v,
                 idx00, idx01, idx10, idx11, idx20, idx21, idx30, idx31,
                 rows_a, rows_b, sem):
    wid = lax.axis_index("s") * 2 + lax.axis_index("c")
    base = wid * (T // 32)
    waits = []
    pltpu.sync_copy(pos_hbm.at[0, pl.ds(base, T // 32)], idxw0_v)
    pltpu.sync_copy(wexp_hbm.at[0, pl.ds(base, T // 32)], w0_v)
    waits.append(pltpu.async_copy(w0_v, ws_hbm.at[idxw0_v], sem))
    pltpu.sync_copy(pos_hbm.at[1, pl.ds(base, T // 32)], idxw1_v)
    pltpu.sync_copy(wexp_hbm.at[1, pl.ds(base, T // 32)], w1_v)
    waits.append(pltpu.async_copy(w1_v, ws_hbm.at[idxw1_v], sem))
    idxs = ((idx00, idx01), (idx10, idx11), (idx20, idx21), (idx30, idx31))
    bufs = (rows_a, rows_b)
    pend = [None, None]
    for c in range(4):
        buf = bufs[c % 2]
        if pend[c % 2] is not None:
            for cp in pend[c % 2]:
                cp.wait()
        cb = base + c * 16
        pltpu.sync_copy(x_hbm.at[pl.ds(cb, 16)], buf)
        pltpu.sync_copy(pos_hbm.at[0, pl.ds(cb, 16)], idxs[c][0])
        pltpu.sync_copy(pos_hbm.at[1, pl.ds(cb, 16)], idxs[c][1])
        pend[c % 2] = (pltpu.async_copy(buf, xs_hbm.at[idxs[c][0]], sem),
                       pltpu.async_copy(buf, xs_hbm.at[idxs[c][1]], sem))
    for pp in pend:
        for cp in pp:
            cp.wait()
    for cp in waits:
        cp.wait()


def _combine_sc(ys_hbm, pos_hbm, out_hbm,
                idx0_all, idx1_all, r0a, r0b, r1a, r1b, outa, outb,
                gsem, wsem):
    wid = lax.axis_index("s") * 2 + lax.axis_index("c")
    base = wid * (T // 32)
    pltpu.sync_copy(pos_hbm.at[0, pl.ds(base, T // 32)], idx0_all)
    pltpu.sync_copy(pos_hbm.at[1, pl.ds(base, T // 32)], idx1_all)
    r0 = (r0a, r0b)
    r1 = (r1a, r1b)
    ob = (outa, outb)
    NCH = 8
    CH = (T // 32) // NCH

    def gath(c):
        sl = pl.ds(c * CH, CH)
        return (pltpu.async_copy(ys_hbm.at[idx0_all.at[sl]], r0[c % 2], gsem),
                pltpu.async_copy(ys_hbm.at[idx1_all.at[sl]], r1[c % 2], gsem))

    pend = gath(0)
    owait = [None, None]
    for c in range(NCH):
        nxt = gath(c + 1) if c + 1 < NCH else None
        for cp in pend:
            cp.wait()
        if owait[c % 2] is not None:
            owait[c % 2].wait()

        def body(j, carry, _c=c):
            sl = pl.ds(j * 16, 16)
            for i in range(CH):
                ob[_c % 2][i, sl] = r0[_c % 2][i, sl] + r1[_c % 2][i, sl]
            return carry

        lax.fori_loop(0, H // 16, body, 0)
        owait[c % 2] = pltpu.async_copy(
            ob[c % 2], out_hbm.at[pl.ds(base + c * CH, CH)], wsem)
        if nxt is not None:
            pend = nxt
    for ow in owait:
        if ow is not None:
            ow.wait()


def _cast_body(wg_ref, wu_ref, wd_ref, og_ref, ou_ref, od_ref):
    og_ref[...] = wg_ref[...].astype(jnp.bfloat16)
    ou_ref[...] = wu_ref[...].astype(jnp.bfloat16)
    od_ref[...] = wd_ref[...].astype(jnp.bfloat16)


def _gemm_body(g_ref, n_ref, xs_ref, wg_ref, wu_ref, wd_ref, ws_ref, ys_ref):
    @pl.when(pl.program_id(0) < n_ref[0])
    def _():
        xb = xs_ref[...].astype(jnp.bfloat16)
        g = jnp.dot(xb, wg_ref[0], preferred_element_type=jnp.float32)
        u = jnp.dot(xb, wu_ref[0], preferred_element_type=jnp.float32)
        a = ((g * jax.nn.sigmoid(g)) * u).astype(jnp.bfloat16)
        y = jnp.dot(a, wd_ref[0], preferred_element_type=jnp.float32)
        ys_ref[...] = y * ws_ref[:, 0:1]


def kernel(hidden_states, router_w, w_gate, w_up, w_down):
    b, s, h = hidden_states.shape
    x = hidden_states.reshape(T, h)

    pos, wexp, toff = pl.pallas_call(
        _router_body,
        out_shape=[
            jax.ShapeDtypeStruct((2, T), jnp.int32),
            jax.ShapeDtypeStruct((2, T, 128), jnp.float32),
            jax.ShapeDtypeStruct((1, 2 * E), jnp.int32),
        ],
    )(x, router_w.T)

    g_arr = jnp.clip(
        jnp.sum(jnp.arange(NJ, dtype=jnp.int32)[:, None] >= toff[0][None, :E], axis=1) - 1,
        0, E - 1).astype(jnp.int32)
    nuse = toff[0, E:E + 1]

    mesh = plsc.VectorSubcoreMesh(core_axis_name="c", subcore_axis_name="s")
    xs, ws = pl.kernel(
        _dispatch_sc,
        mesh=mesh,
        out_type=[
            jax.ShapeDtypeStruct((NS, H), jnp.float32),
            jax.ShapeDtypeStruct((NS, 128), jnp.float32),
        ],
        scratch_types=(
            [pltpu.VMEM((T // 32,), jnp.int32)] * 2
            + [pltpu.VMEM((T // 32, 128), jnp.float32)] * 2
            + [pltpu.VMEM((16,), jnp.int32)] * 8
            + [pltpu.VMEM((16, H), jnp.float32)] * 2
            + [pltpu.SemaphoreType.DMA]
        ),
    )(x, pos, wexp)

    wg16, wu16, wd16 = pl.pallas_call(
        _cast_body,
        grid=(E, F // 128),
        in_specs=[
            pl.BlockSpec((1, H, 128), lambda e, f: (e, 0, f)),
            pl.BlockSpec((1, H, 128), lambda e, f: (e, 0, f)),
            pl.BlockSpec((1, 128, H), lambda e, f: (e, f, 0)),
        ],
        out_specs=[
            pl.BlockSpec((1, H, 128), lambda e, f: (e, 0, f)),
            pl.BlockSpec((1, H, 128), lambda e, f: (e, 0, f)),
            pl.BlockSpec((1, 128, H), lambda e, f: (e, f, 0)),
        ],
        out_shape=[
            jax.ShapeDtypeStruct((E, H, F), jnp.bfloat16),
            jax.ShapeDtypeStruct((E, H, F), jnp.bfloat16),
            jax.ShapeDtypeStruct((E, F, H), jnp.bfloat16),
        ],
        compiler_params=pltpu.CompilerParams(
            dimension_semantics=("arbitrary", "arbitrary"),
        ),
    )(w_gate, w_up, w_down)

    grid_spec = pltpu.PrefetchScalarGridSpec(
        num_scalar_prefetch=2,
        grid=(NJ,),
        in_specs=[
            pl.BlockSpec((TM, H), lambda j, g, n: (j, 0)),
            pl.BlockSpec((1, H, F), lambda j, g, n: (g[j], 0, 0)),
            pl.BlockSpec((1, H, F), lambda j, g, n: (g[j], 0, 0)),
            pl.BlockSpec((1, F, H), lambda j, g, n: (g[j], 0, 0)),
            pl.BlockSpec((TM, 128), lambda j, g, n: (j, 0)),
        ],
        out_specs=pl.BlockSpec((TM, H), lambda j, g, n: (j, 0)),
    )
    ys = pl.pallas_call(
        _gemm_body,
        grid_spec=grid_spec,
        out_shape=jax.ShapeDtypeStruct((NS, H), jnp.float32),
        compiler_params=pltpu.CompilerParams(
            dimension_semantics=("arbitrary",),
        ),
    )(g_arr, nuse, xs, wg16, wu16, wd16, ws)

    out = pl.kernel(
        _combine_sc,
        mesh=mesh,
        out_type=jax.ShapeDtypeStruct((T, H), jnp.float32),
        scratch_types=(
            [pltpu.VMEM((T // 32,), jnp.int32)] * 2
            + [pltpu.VMEM((8, H), jnp.float32)] * 6
            + [pltpu.SemaphoreType.DMA, pltpu.SemaphoreType.DMA]
        ),
    )(ys, pos)
    return out.reshape(b, s, h)
